# baseline (speedup 1.0000x reference)
import functools

import jax
import jax.numpy as jnp
from jax.experimental import pallas as pl
from jax.experimental.pallas import tpu as pltpu

K = 1024
D = 64
CC = 0.25
NRQ = 3
N_TOTAL = 16 * 1024
BN = 512
N_STEPS = N_TOTAL // BN


def _rowsum64(a):
    n = a.shape[0]
    b = a.reshape(n, 8, 8)
    s = b[:, 0, :]
    for i in range(1, 8):
        s = s + b[:, i, :]
    t = s[:, 0:4] + s[:, 4:8]
    u = t[:, 0:2] + t[:, 2:4]
    return u[:, 0:1] + u[:, 1:2]


def _rvq_kernel(x_ref, w0_ref, w1_ref, w2_ref, w3_ref,
                loss_ref, quant_ref, perp_ref, enc_ref,
                hist_scr, sse_scr):
    i = pl.program_id(0)

    @pl.when(i == 0)
    def _init():
        hist_scr[...] = jnp.zeros_like(hist_scr)
        sse_scr[...] = jnp.zeros_like(sse_scr)

    x = x_ref[...]
    quant = jnp.zeros_like(x)
    sse = sse_scr[...]
    hist = hist_scr[...]
    iota_k = jax.lax.broadcasted_iota(jnp.int32, (BN, K), 1)

    last_onehot = None
    for w_ref in (w0_ref, w1_ref, w2_ref, w3_ref):
        w = w_ref[...]
        res = x - quant
        rn = _rowsum64(res * res)
        cn = _rowsum64(w * w).reshape(K)[None, :]
        xw = jax.lax.dot_general(
            res, w, (((1,), (1,)), ((), ())),
            preferred_element_type=jnp.float32)
        dist = rn + cn - 2.0 * xw
        dmin = jnp.min(dist, axis=1, keepdims=True)
        cand = jnp.where(dist == dmin, iota_k, K)
        idx = jnp.min(cand, axis=1, keepdims=True)
        onehot = (iota_k == idx).astype(jnp.float32)
        q = jax.lax.dot_general(
            onehot, w, (((1,), (0,)), ((), ())),
            preferred_element_type=jnp.float32)
        err = q - res
        sse = sse + jnp.sum(err * err).reshape(1, 1)
        hist = hist + jnp.sum(onehot, axis=0, keepdims=True)
        quant = quant + q
        last_onehot = onehot

    sse_scr[...] = sse
    hist_scr[...] = hist
    quant_ref[...] = x + (quant - x)
    enc_ref[...] = last_onehot

    @pl.when(i == N_STEPS - 1)
    def _finish():
        total = sse_scr[0, 0]
        loss_ref[...] = ((1.0 + CC) * total / (N_TOTAL * D)).reshape(1, 1)
        p = hist_scr[...] / (N_TOTAL * (NRQ + 1))
        ent = -jnp.sum(p * jnp.log(p + 1e-10))
        perp_ref[...] = jnp.exp(ent).reshape(1, 1)


@jax.jit
def kernel(inputs, emb_w, res_w0, res_w1, res_w2):
    input_shape = inputs.shape
    flat = inputs.reshape(-1, D)

    out_shapes = (
        jax.ShapeDtypeStruct((1, 1), jnp.float32),
        jax.ShapeDtypeStruct((N_TOTAL, D), jnp.float32),
        jax.ShapeDtypeStruct((1, 1), jnp.float32),
        jax.ShapeDtypeStruct((N_TOTAL, K), jnp.float32),
    )
    w_spec = pl.BlockSpec((K, D), lambda i: (0, 0))
    loss, quant, perp, enc = pl.pallas_call(
        _rvq_kernel,
        grid=(N_STEPS,),
        in_specs=[
            pl.BlockSpec((BN, D), lambda i: (i, 0)),
            w_spec, w_spec, w_spec, w_spec,
        ],
        out_specs=(
            pl.BlockSpec((1, 1), lambda i: (0, 0)),
            pl.BlockSpec((BN, D), lambda i: (i, 0)),
            pl.BlockSpec((1, 1), lambda i: (0, 0)),
            pl.BlockSpec((BN, K), lambda i: (i, 0)),
        ),
        out_shape=out_shapes,
        scratch_shapes=[
            pltpu.VMEM((1, K), jnp.float32),
            pltpu.VMEM((1, 1), jnp.float32),
        ],
    )(flat, emb_w, res_w0, res_w1, res_w2)

    return (loss[0, 0], quant.reshape(input_shape), perp[0, 0], enc)


# speedup vs baseline: 5.1356x; 5.1356x over previous
import jax
import jax.numpy as jnp
from jax.experimental import pallas as pl
from jax.experimental.pallas import tpu as pltpu

K = 1024
D = 64
CC = 0.25
NRQ = 3
N_TOTAL = 16 * 1024
BN = 512
N_STEPS = N_TOTAL // BN


def _rowsum64(a):
    acc = a
    for i in range(1, 8):
        acc = acc + pltpu.roll(a, 64 - 8 * i, 1)
    t = acc[:, 0:4] + acc[:, 4:8]
    u = t[:, 0:2] + t[:, 2:4]
    return u[:, 0:1] + u[:, 1:2]


def _colsum64(a):
    acc = a[0:8, :]
    for i in range(1, 8):
        acc = acc + a[8 * i:8 * i + 8, :]
    t = acc[0:4, :] + acc[4:8, :]
    u = t[0:2, :] + t[2:4, :]
    return u[0:1, :] + u[1:2, :]


def _rvq_kernel(x_ref, w0_ref, w1_ref, w2_ref, w3_ref,
                wt0_ref, wt1_ref, wt2_ref, wt3_ref,
                loss_ref, quant_ref, perp_ref, enc_ref,
                cn_scr, hist_scr, sse_scr):
    i = pl.program_id(0)

    @pl.when(i == 0)
    def _init():
        hist_scr[...] = jnp.zeros_like(hist_scr)
        sse_scr[...] = jnp.zeros_like(sse_scr)
        for s, wt_ref in enumerate((wt0_ref, wt1_ref, wt2_ref, wt3_ref)):
            wt = wt_ref[...]
            cn_scr[s:s + 1, :] = _colsum64(wt * wt)

    x = x_ref[...]
    quant = jnp.zeros_like(x)
    sse = sse_scr[...]
    hist = hist_scr[...]
    iota_k = jax.lax.broadcasted_iota(jnp.int32, (BN, K), 1)

    last_onehot = None
    for s, w_ref in enumerate((w0_ref, w1_ref, w2_ref, w3_ref)):
        w = w_ref[...]
        res = x - quant
        rn = _rowsum64(res * res)
        cn = cn_scr[s:s + 1, :]
        xw = jax.lax.dot_general(
            res, w, (((1,), (1,)), ((), ())),
            preferred_element_type=jnp.float32)
        dist = rn + cn - 2.0 * xw
        dmin = jnp.min(dist, axis=1, keepdims=True)
        cand = jnp.where(dist == dmin, iota_k, K)
        idx = jnp.min(cand, axis=1, keepdims=True)
        onehot = (iota_k == idx).astype(jnp.float32)
        q = jax.lax.dot_general(
            onehot, w, (((1,), (0,)), ((), ())),
            preferred_element_type=jnp.float32)
        err = q - res
        sse = sse + jnp.sum(err * err).reshape(1, 1)
        hist = hist + jnp.sum(onehot, axis=0, keepdims=True)
        quant = quant + q
        last_onehot = onehot

    sse_scr[...] = sse
    hist_scr[...] = hist
    quant_ref[...] = x + (quant - x)
    enc_ref[...] = last_onehot

    @pl.when(i == N_STEPS - 1)
    def _finish():
        total = sse_scr[0, 0]
        loss_ref[...] = ((1.0 + CC) * total / (N_TOTAL * D)).reshape(1, 1)
        p = hist_scr[...] / (N_TOTAL * (NRQ + 1))
        ent = -jnp.sum(p * jnp.log(p + 1e-10))
        perp_ref[...] = jnp.exp(ent).reshape(1, 1)


@jax.jit
def kernel(inputs, emb_w, res_w0, res_w1, res_w2):
    input_shape = inputs.shape
    flat = inputs.reshape(-1, D)

    out_shapes = (
        jax.ShapeDtypeStruct((1, 1), jnp.float32),
        jax.ShapeDtypeStruct((N_TOTAL, D), jnp.float32),
        jax.ShapeDtypeStruct((1, 1), jnp.float32),
        jax.ShapeDtypeStruct((N_TOTAL, K), jnp.float32),
    )
    w_spec = pl.BlockSpec((K, D), lambda i: (0, 0))
    wt_spec = pl.BlockSpec((D, K), lambda i: (0, 0))
    loss, quant, perp, enc = pl.pallas_call(
        _rvq_kernel,
        grid=(N_STEPS,),
        in_specs=[
            pl.BlockSpec((BN, D), lambda i: (i, 0)),
            w_spec, w_spec, w_spec, w_spec,
            wt_spec, wt_spec, wt_spec, wt_spec,
        ],
        out_specs=(
            pl.BlockSpec((1, 1), lambda i: (0, 0)),
            pl.BlockSpec((BN, D), lambda i: (i, 0)),
            pl.BlockSpec((1, 1), lambda i: (0, 0)),
            pl.BlockSpec((BN, K), lambda i: (i, 0)),
        ),
        out_shape=out_shapes,
        scratch_shapes=[
            pltpu.VMEM((4, K), jnp.float32),
            pltpu.VMEM((1, K), jnp.float32),
            pltpu.VMEM((1, 1), jnp.float32),
        ],
    )(flat, emb_w, res_w0, res_w1, res_w2,
      emb_w.T, res_w0.T, res_w1.T, res_w2.T)

    return (loss[0, 0], quant.reshape(input_shape), perp[0, 0], enc)


# speedup vs baseline: 5.5825x; 1.0870x over previous
import jax
import jax.numpy as jnp
from jax.experimental import pallas as pl
from jax.experimental.pallas import tpu as pltpu

K = 1024
D = 64
CC = 0.25
NRQ = 3
N_TOTAL = 16 * 1024
BN = 1024
N_STEPS = N_TOTAL // BN


def _rowsum64(a):
    acc = a
    for i in range(1, 8):
        acc = acc + pltpu.roll(a, 64 - 8 * i, 1)
    t = acc[:, 0:4] + acc[:, 4:8]
    u = t[:, 0:2] + t[:, 2:4]
    return u[:, 0:1] + u[:, 1:2]


def _colsum64(a):
    acc = a[0:8, :]
    for i in range(1, 8):
        acc = acc + a[8 * i:8 * i + 8, :]
    t = acc[0:4, :] + acc[4:8, :]
    u = t[0:2, :] + t[2:4, :]
    return u[0:1, :] + u[1:2, :]


def _rvq_kernel(x_ref, w0_ref, w1_ref, w2_ref, w3_ref,
                wt0_ref, wt1_ref, wt2_ref, wt3_ref,
                loss_ref, quant_ref, perp_ref, enc_ref,
                cn_scr, hist_scr, sse_scr):
    i = pl.program_id(0)

    @pl.when(i == 0)
    def _init():
        hist_scr[...] = jnp.zeros_like(hist_scr)
        sse_scr[...] = jnp.zeros_like(sse_scr)
        for s, wt_ref in enumerate((wt0_ref, wt1_ref, wt2_ref, wt3_ref)):
            wt = wt_ref[...]
            cn_scr[s:s + 1, :] = _colsum64(wt * wt)

    x = x_ref[...]
    quant = jnp.zeros_like(x)
    sse = sse_scr[...]
    hist = hist_scr[...]
    iota_k = jax.lax.broadcasted_iota(jnp.int32, (BN, K), 1)

    last_onehot = None
    for s, w_ref in enumerate((w0_ref, w1_ref, w2_ref, w3_ref)):
        w = w_ref[...]
        res = x - quant
        rn = _rowsum64(res * res)
        cn = cn_scr[s:s + 1, :]
        xw = jax.lax.dot_general(
            res, w, (((1,), (1,)), ((), ())),
            preferred_element_type=jnp.float32)
        dist = rn + cn - 2.0 * xw
        dmin = jnp.min(dist, axis=1, keepdims=True)
        cand = jnp.where(dist == dmin, iota_k, K)
        idx = jnp.min(cand, axis=1, keepdims=True)
        onehot = (iota_k == idx).astype(jnp.float32)
        q = jax.lax.dot_general(
            onehot, w, (((1,), (0,)), ((), ())),
            preferred_element_type=jnp.float32)
        err = q - res
        sse = sse + jnp.sum(err * err).reshape(1, 1)
        hist = hist + jnp.sum(onehot, axis=0, keepdims=True)
        quant = quant + q
        last_onehot = onehot

    sse_scr[...] = sse
    hist_scr[...] = hist
    quant_ref[...] = x + (quant - x)
    enc_ref[...] = last_onehot

    @pl.when(i == N_STEPS - 1)
    def _finish():
        total = sse_scr[0, 0]
        loss_ref[...] = ((1.0 + CC) * total / (N_TOTAL * D)).reshape(1, 1)
        p = hist_scr[...] / (N_TOTAL * (NRQ + 1))
        ent = -jnp.sum(p * jnp.log(p + 1e-10))
        perp_ref[...] = jnp.exp(ent).reshape(1, 1)


@jax.jit
def kernel(inputs, emb_w, res_w0, res_w1, res_w2):
    input_shape = inputs.shape
    flat = inputs.reshape(-1, D)

    out_shapes = (
        jax.ShapeDtypeStruct((1, 1), jnp.float32),
        jax.ShapeDtypeStruct((N_TOTAL, D), jnp.float32),
        jax.ShapeDtypeStruct((1, 1), jnp.float32),
        jax.ShapeDtypeStruct((N_TOTAL, K), jnp.float32),
    )
    w_spec = pl.BlockSpec((K, D), lambda i: (0, 0))
    wt_spec = pl.BlockSpec((D, K), lambda i: (0, 0))
    loss, quant, perp, enc = pl.pallas_call(
        _rvq_kernel,
        grid=(N_STEPS,),
        in_specs=[
            pl.BlockSpec((BN, D), lambda i: (i, 0)),
            w_spec, w_spec, w_spec, w_spec,
            wt_spec, wt_spec, wt_spec, wt_spec,
        ],
        out_specs=(
            pl.BlockSpec((1, 1), lambda i: (0, 0)),
            pl.BlockSpec((BN, D), lambda i: (i, 0)),
            pl.BlockSpec((1, 1), lambda i: (0, 0)),
            pl.BlockSpec((BN, K), lambda i: (i, 0)),
        ),
        out_shape=out_shapes,
        scratch_shapes=[
            pltpu.VMEM((4, K), jnp.float32),
            pltpu.VMEM((1, K), jnp.float32),
            pltpu.VMEM((1, 1), jnp.float32),
        ],
    )(flat, emb_w, res_w0, res_w1, res_w2,
      emb_w.T, res_w0.T, res_w1.T, res_w2.T)

    return (loss[0, 0], quant.reshape(input_shape), perp[0, 0], enc)
